# baseline (speedup 1.0000x reference)
import functools

import jax
import jax.numpy as jnp
import numpy as np
from jax import lax
from jax.experimental import pallas as pl
from jax.experimental.pallas import tpu as pltpu
from jax.experimental.pallas import tpu_sc as plsc

_N = 10000
_E = 320000
_M = 128
_HID = 128
_MY = 40
_LAYERS = 16
_ALPHA = 0.1
_THETA = 0.5

_NC = 2
_NS = 16
_LANES = 16
_NW = _NC * _NS
_CHUNK = 128
_NB = 2
_NSTEPS = 80
_NSTAGE = 2
_HSTEPS = _NSTEPS // _NSTAGE
_NG = _HSTEPS // _NB
_EP = _NW * _NSTEPS * _CHUNK
_AGG_ROWS = 10112
_STRIPE = _AGG_ROWS // _NS
_ZR = 32
_DEGW = 128

_BR = 1000


def _sc_edge_scatter(hs, srcw, dstw, width):
    mesh = plsc.VectorSubcoreMesh(core_axis_name="c", subcore_axis_name="s")

    def body(hs_hbm, srcw_hbm, dstw_hbm, out_hbm, agg_sh, idx_s, idx_d, *bufs):
        rows = list(bufs[0:_NB])
        gsem = list(bufs[_NB:2 * _NB])
        ssem = list(bufs[2 * _NB:3 * _NB])
        zbuf = bufs[3 * _NB]
        cid = lax.axis_index("c")
        sid = lax.axis_index("s")
        wid = cid * _NS + sid

        def gwait(b):
            pltpu.make_async_copy(hs_hbm.at[pl.ds(0, _CHUNK)], rows[b], gsem[b]).wait()

        def swait(b):
            pltpu.make_async_copy(hs_hbm.at[pl.ds(0, _CHUNK)], rows[b], ssem[b]).wait()

        pltpu.sync_copy(srcw_hbm.at[wid, pl.ds(0, _HSTEPS)], idx_s)
        pltpu.sync_copy(dstw_hbm.at[wid, pl.ds(0, _HSTEPS)], idx_d)
        for b in range(_NB):
            pltpu.async_copy(hs_hbm.at[idx_s.at[b]], rows[b], gsem[b])

        def zrow(r, carry):
            for c in range(width // _LANES):
                zbuf[r, pl.ds(c * _LANES, _LANES)] = jnp.zeros((_LANES,), jnp.float32)
            return carry
        lax.fori_loop(0, _ZR, zrow, 0)
        full, rem = divmod(_STRIPE, _ZR)
        for k in range(full):
            pltpu.sync_copy(zbuf, agg_sh.at[pl.ds(sid * _STRIPE + k * _ZR, _ZR)])
        if rem:
            pltpu.sync_copy(zbuf.at[pl.ds(0, rem)],
                            agg_sh.at[pl.ds(sid * _STRIPE + full * _ZR, rem)])
        plsc.subcore_barrier()

        for stage in range(_NSTAGE):
            def group(g, carry):
                for b in range(_NB):
                    gwait(b)
                    pltpu.async_copy(rows[b], agg_sh.at[idx_d.at[_NB * g + b]],
                                     ssem[b], add=True)
                for b in range(_NB):
                    @pl.when(g < _NG - 1)
                    def _():
                        swait(b)
                        pltpu.async_copy(hs_hbm.at[idx_s.at[_NB * (g + 1) + b]],
                                         rows[b], gsem[b])
                return carry

            lax.fori_loop(0, _NG, group, 0)
            for b in range(_NB):
                swait(b)
            if stage < _NSTAGE - 1:
                pltpu.sync_copy(srcw_hbm.at[wid, pl.ds((stage + 1) * _HSTEPS, _HSTEPS)], idx_s)
                pltpu.sync_copy(dstw_hbm.at[wid, pl.ds((stage + 1) * _HSTEPS, _HSTEPS)], idx_d)
                for b in range(_NB):
                    pltpu.async_copy(hs_hbm.at[idx_s.at[b]], rows[b], gsem[b])
        plsc.subcore_barrier()

        pltpu.sync_copy(agg_sh.at[pl.ds(sid * _STRIPE, _STRIPE)],
                        out_hbm.at[cid, pl.ds(sid * _STRIPE, _STRIPE)])

    scratch = [
        pltpu.VMEM_SHARED((_AGG_ROWS, width), jnp.float32),
        pltpu.VMEM((_HSTEPS, _CHUNK), jnp.int32),
        pltpu.VMEM((_HSTEPS, _CHUNK), jnp.int32),
    ]
    scratch += [pltpu.VMEM((_CHUNK, width), jnp.float32)] * _NB
    scratch += [pltpu.SemaphoreType.DMA] * (2 * _NB)
    scratch += [pltpu.VMEM((_ZR, width), jnp.float32)]

    return pl.kernel(
        body,
        out_type=jax.ShapeDtypeStruct((_NC, _AGG_ROWS, width), jnp.float32),
        mesh=mesh,
        scratch_types=scratch,
    )(hs, srcw, dstw)


def _tc_init(x, W1, b1, D):
    def body(x_ref, w_ref, b_ref, d_ref, h_ref, hs_ref, dinv_ref):
        h = jnp.dot(x_ref[...], w_ref[...], preferred_element_type=jnp.float32)
        h = jnp.maximum(h + b_ref[...], 0.0)
        dnarrow = d_ref[0] + d_ref[1] + 1.0
        deg = jnp.broadcast_to(dnarrow[:, 0:1], (_BR, _HID))
        dinv = lax.rsqrt(deg)
        h_ref[...] = h
        hs_ref[...] = h * dinv
        dinv_ref[...] = dinv

    return pl.pallas_call(
        body,
        grid=(_N // _BR,),
        in_specs=[
            pl.BlockSpec((_BR, _M), lambda i: (i, 0)),
            pl.BlockSpec((_M, _HID), lambda i: (0, 0)),
            pl.BlockSpec((1, _HID), lambda i: (0, 0)),
            pl.BlockSpec((_NC, _BR, _DEGW), lambda i: (0, i, 0)),
        ],
        out_specs=[
            pl.BlockSpec((_BR, _HID), lambda i: (i, 0)),
            pl.BlockSpec((_BR, _HID), lambda i: (i, 0)),
            pl.BlockSpec((_BR, _HID), lambda i: (i, 0)),
        ],
        out_shape=[
            jax.ShapeDtypeStruct((_N, _HID), jnp.float32),
            jax.ShapeDtypeStruct((_N, _HID), jnp.float32),
            jax.ShapeDtypeStruct((_N, _HID), jnp.float32),
        ],
    )(x, W1, b1.reshape(1, _HID), D)


def _tc_layer(S, hs, h0, dinv, weff_i):
    def body(s_ref, hs_ref, h0_ref, dinv_ref, w_ref, h_ref, hs_out_ref):
        ssum = s_ref[0] + s_ref[1] + hs_ref[...]
        supp = (1.0 - _ALPHA) * dinv_ref[...] * ssum + _ALPHA * h0_ref[...]
        h = jnp.dot(supp, w_ref[...], preferred_element_type=jnp.float32)
        h = jnp.maximum(h, 0.0)
        h_ref[...] = h
        hs_out_ref[...] = h * dinv_ref[...]

    return pl.pallas_call(
        body,
        grid=(_N // _BR,),
        in_specs=[
            pl.BlockSpec((_NC, _BR, _HID), lambda i: (0, i, 0)),
            pl.BlockSpec((_BR, _HID), lambda i: (i, 0)),
            pl.BlockSpec((_BR, _HID), lambda i: (i, 0)),
            pl.BlockSpec((_BR, _HID), lambda i: (i, 0)),
            pl.BlockSpec((_HID, _HID), lambda i: (0, 0)),
        ],
        out_specs=[
            pl.BlockSpec((_BR, _HID), lambda i: (i, 0)),
            pl.BlockSpec((_BR, _HID), lambda i: (i, 0)),
        ],
        out_shape=[
            jax.ShapeDtypeStruct((_N, _HID), jnp.float32),
            jax.ShapeDtypeStruct((_N, _HID), jnp.float32),
        ],
    )(S, hs, h0, dinv, weff_i)


def _tc_final(h, W2, b2):
    def body(h_ref, w_ref, b_ref, o_ref):
        o = jnp.dot(h_ref[...], w_ref[...], preferred_element_type=jnp.float32)
        o_ref[...] = o + b_ref[...]

    return pl.pallas_call(
        body,
        grid=(_N // _BR,),
        in_specs=[
            pl.BlockSpec((_BR, _HID), lambda i: (i, 0)),
            pl.BlockSpec((_HID, _MY), lambda i: (0, 0)),
            pl.BlockSpec((1, _MY), lambda i: (0, 0)),
        ],
        out_specs=pl.BlockSpec((_BR, _MY), lambda i: (i, 0)),
        out_shape=jax.ShapeDtypeStruct((_N, _MY), jnp.float32),
    )(h, W2, b2.reshape(1, _MY))


def kernel(x, edge_index, W1, b1, conv_w, W2, b2):
    src = edge_index[0].astype(jnp.int32)
    dst = edge_index[1].astype(jnp.int32)
    pad = _EP - _E
    pad_src = jnp.arange(pad, dtype=jnp.int32) % _N
    pad_dst = _N + (jnp.arange(pad, dtype=jnp.int32) % (_AGG_ROWS - _N))
    srcw = jnp.concatenate([src, pad_src]).reshape(_NW, _NSTEPS, _CHUNK)
    dstw = jnp.concatenate([dst, pad_dst]).reshape(_NW, _NSTEPS, _CHUNK)

    eye = jnp.eye(_HID, dtype=jnp.float32)
    betas = [float(np.log(_THETA / (i + 1) + 1.0)) for i in range(_LAYERS)]
    weff = [(1.0 - b) * eye + b * conv_w[i] for i, b in enumerate(betas)]

    ones = jnp.ones((_N, _DEGW), jnp.float32)
    D = _sc_edge_scatter(ones, srcw, dstw, _DEGW)
    h0, hs, dinv = _tc_init(x, W1, b1, D)

    h = h0
    for i in range(_LAYERS):
        S = _sc_edge_scatter(hs, srcw, dstw, _HID)
        h, hs = _tc_layer(S, hs, h0, dinv, weff[i])
    return _tc_final(h, W2, b2)


# speedup vs baseline: 1.2502x; 1.2502x over previous
import functools

import jax
import jax.numpy as jnp
import numpy as np
from jax import lax
from jax.experimental import pallas as pl
from jax.experimental.pallas import tpu as pltpu
from jax.experimental.pallas import tpu_sc as plsc

_N = 10000
_E = 320000
_M = 128
_HID = 128
_MY = 40
_LAYERS = 16
_ALPHA = 0.1
_THETA = 0.5

_NC = 2
_NS = 16
_LANES = 16
_NW = _NC * _NS
_CHUNK = 128
_NSTEPS = 80
_HSTEPS = _NSTEPS // 2
_EP = _NW * _NSTEPS * _CHUNK
_AGG_ROWS = 10112
_STRIPE = _AGG_ROWS // _NS
_ZR = 32

_BR = 1000


def _sc_edge_scatter(hs, srcw, dstw):
    mesh = plsc.VectorSubcoreMesh(core_axis_name="c", subcore_axis_name="s")

    def body(hs_hbm, srcw_hbm, dstw_hbm, out_hbm,
             agg_sh, idx_s, idx_d, rows0, rows1, zbuf, gsem0, gsem1):
        cid = lax.axis_index("c")
        sid = lax.axis_index("s")
        wid = cid * _NS + sid

        pltpu.sync_copy(srcw_hbm.at[wid, pl.ds(0, _HSTEPS)], idx_s)
        pltpu.sync_copy(dstw_hbm.at[wid, pl.ds(0, _HSTEPS)], idx_d)
        pltpu.async_copy(hs_hbm.at[idx_s.at[0]], rows0, gsem0)

        def zrow(r, carry):
            for c in range(_HID // _LANES):
                zbuf[r, pl.ds(c * _LANES, _LANES)] = jnp.zeros((_LANES,), jnp.float32)
            return carry
        lax.fori_loop(0, _ZR, zrow, 0)
        full, rem = divmod(_STRIPE, _ZR)
        for k in range(full):
            pltpu.sync_copy(zbuf, agg_sh.at[pl.ds(sid * _STRIPE + k * _ZR, _ZR)])
        if rem:
            pltpu.sync_copy(zbuf.at[pl.ds(0, rem)],
                            agg_sh.at[pl.ds(sid * _STRIPE + full * _ZR, rem)])
        plsc.subcore_barrier()

        for half in range(_NSTEPS // _HSTEPS):
            if half > 0:
                pltpu.sync_copy(srcw_hbm.at[wid, pl.ds(half * _HSTEPS, _HSTEPS)], idx_s)
                pltpu.sync_copy(dstw_hbm.at[wid, pl.ds(half * _HSTEPS, _HSTEPS)], idx_d)
                pltpu.async_copy(hs_hbm.at[idx_s.at[0]], rows0, gsem0)

            def step(i, carry):
                pltpu.make_async_copy(hs_hbm.at[pl.ds(0, _CHUNK)], rows0, gsem0).wait()
                pltpu.async_copy(hs_hbm.at[idx_s.at[2 * i + 1]], rows1, gsem1)
                pltpu.sync_copy(rows0, agg_sh.at[idx_d.at[2 * i]], add=True)

                @pl.when(i < _HSTEPS // 2 - 1)
                def _():
                    pltpu.async_copy(hs_hbm.at[idx_s.at[2 * i + 2]], rows0, gsem0)

                pltpu.make_async_copy(hs_hbm.at[pl.ds(0, _CHUNK)], rows1, gsem1).wait()
                pltpu.sync_copy(rows1, agg_sh.at[idx_d.at[2 * i + 1]], add=True)
                return carry

            lax.fori_loop(0, _HSTEPS // 2, step, 0)
        plsc.subcore_barrier()

        pltpu.sync_copy(agg_sh.at[pl.ds(sid * _STRIPE, _STRIPE)],
                        out_hbm.at[cid, pl.ds(sid * _STRIPE, _STRIPE)])

    return pl.kernel(
        body,
        out_type=jax.ShapeDtypeStruct((_NC, _AGG_ROWS, _HID), jnp.float32),
        mesh=mesh,
        scratch_types=[
            pltpu.VMEM_SHARED((_AGG_ROWS, _HID), jnp.float32),
            pltpu.VMEM((_HSTEPS, _CHUNK), jnp.int32),
            pltpu.VMEM((_HSTEPS, _CHUNK), jnp.int32),
            pltpu.VMEM((_CHUNK, _HID), jnp.float32),
            pltpu.VMEM((_CHUNK, _HID), jnp.float32),
            pltpu.VMEM((_ZR, _HID), jnp.float32),
            pltpu.SemaphoreType.DMA,
            pltpu.SemaphoreType.DMA,
        ],
    )(hs, srcw, dstw)


def _tc_h0(x, W1, b1):
    def body(x_ref, w_ref, b_ref, h_ref):
        h = jnp.dot(x_ref[...], w_ref[...], preferred_element_type=jnp.float32)
        h_ref[...] = jnp.maximum(h + b_ref[...], 0.0)

    return pl.pallas_call(
        body,
        grid=(_N // _BR,),
        in_specs=[
            pl.BlockSpec((_BR, _M), lambda i: (i, 0)),
            pl.BlockSpec((_M, _HID), lambda i: (0, 0)),
            pl.BlockSpec((1, _HID), lambda i: (0, 0)),
        ],
        out_specs=pl.BlockSpec((_BR, _HID), lambda i: (i, 0)),
        out_shape=jax.ShapeDtypeStruct((_N, _HID), jnp.float32),
    )(x, W1, b1.reshape(1, _HID))


def _tc_scale(h0, D):
    def body(h_ref, d_ref, hs_ref, dinv_ref):
        deg = d_ref[0] + d_ref[1] + 1.0
        dinv = lax.rsqrt(deg)
        hs_ref[...] = h_ref[...] * dinv
        dinv_ref[...] = dinv

    return pl.pallas_call(
        body,
        grid=(_N // _BR,),
        in_specs=[
            pl.BlockSpec((_BR, _HID), lambda i: (i, 0)),
            pl.BlockSpec((_NC, _BR, _HID), lambda i: (0, i, 0)),
        ],
        out_specs=[
            pl.BlockSpec((_BR, _HID), lambda i: (i, 0)),
            pl.BlockSpec((_BR, _HID), lambda i: (i, 0)),
        ],
        out_shape=[
            jax.ShapeDtypeStruct((_N, _HID), jnp.float32),
            jax.ShapeDtypeStruct((_N, _HID), jnp.float32),
        ],
    )(h0, D)


def _tc_layer(S, hs, h0, dinv, weff_i):
    def body(s_ref, hs_ref, h0_ref, dinv_ref, w_ref, h_ref, hs_out_ref):
        ssum = s_ref[0] + s_ref[1] + hs_ref[...]
        supp = (1.0 - _ALPHA) * dinv_ref[...] * ssum + _ALPHA * h0_ref[...]
        h = jnp.dot(supp, w_ref[...], preferred_element_type=jnp.float32)
        h = jnp.maximum(h, 0.0)
        h_ref[...] = h
        hs_out_ref[...] = h * dinv_ref[...]

    return pl.pallas_call(
        body,
        grid=(_N // _BR,),
        in_specs=[
            pl.BlockSpec((_NC, _BR, _HID), lambda i: (0, i, 0)),
            pl.BlockSpec((_BR, _HID), lambda i: (i, 0)),
            pl.BlockSpec((_BR, _HID), lambda i: (i, 0)),
            pl.BlockSpec((_BR, _HID), lambda i: (i, 0)),
            pl.BlockSpec((_HID, _HID), lambda i: (0, 0)),
        ],
        out_specs=[
            pl.BlockSpec((_BR, _HID), lambda i: (i, 0)),
            pl.BlockSpec((_BR, _HID), lambda i: (i, 0)),
        ],
        out_shape=[
            jax.ShapeDtypeStruct((_N, _HID), jnp.float32),
            jax.ShapeDtypeStruct((_N, _HID), jnp.float32),
        ],
    )(S, hs, h0, dinv, weff_i)


def _tc_final(h, W2, b2):
    def body(h_ref, w_ref, b_ref, o_ref):
        o = jnp.dot(h_ref[...], w_ref[...], preferred_element_type=jnp.float32)
        o_ref[...] = o + b_ref[...]

    return pl.pallas_call(
        body,
        grid=(_N // _BR,),
        in_specs=[
            pl.BlockSpec((_BR, _HID), lambda i: (i, 0)),
            pl.BlockSpec((_HID, _MY), lambda i: (0, 0)),
            pl.BlockSpec((1, _MY), lambda i: (0, 0)),
        ],
        out_specs=pl.BlockSpec((_BR, _MY), lambda i: (i, 0)),
        out_shape=jax.ShapeDtypeStruct((_N, _MY), jnp.float32),
    )(h, W2, b2.reshape(1, _MY))


def kernel(x, edge_index, W1, b1, conv_w, W2, b2):
    src = edge_index[0].astype(jnp.int32)
    dst = edge_index[1].astype(jnp.int32)
    pad = _EP - _E
    pad_src = jnp.arange(pad, dtype=jnp.int32) % _N
    pad_dst = _N + (jnp.arange(pad, dtype=jnp.int32) % (_AGG_ROWS - _N))
    srcw = jnp.concatenate([src, pad_src]).reshape(_NW, _NSTEPS, _CHUNK)
    dstw = jnp.concatenate([dst, pad_dst]).reshape(_NW, _NSTEPS, _CHUNK)

    eye = jnp.eye(_HID, dtype=jnp.float32)
    betas = [float(np.log(_THETA / (i + 1) + 1.0)) for i in range(_LAYERS)]
    weff = [(1.0 - b) * eye + b * conv_w[i] for i, b in enumerate(betas)]

    ones = jnp.ones((_N, _HID), jnp.float32)
    D = _sc_edge_scatter(ones, srcw, dstw)
    h0 = _tc_h0(x, W1, b1)
    hs, dinv = _tc_scale(h0, D)

    h = h0
    for i in range(_LAYERS):
        S = _sc_edge_scatter(hs, srcw, dstw)
        h, hs = _tc_layer(S, hs, h0, dinv, weff[i])
    return _tc_final(h, W2, b2)


# speedup vs baseline: 1.2639x; 1.0110x over previous
import functools

import jax
import jax.numpy as jnp
import numpy as np
from jax import lax
from jax.experimental import pallas as pl
from jax.experimental.pallas import tpu as pltpu
from jax.experimental.pallas import tpu_sc as plsc

_N = 10000
_E = 320000
_M = 128
_HID = 128
_MY = 40
_LAYERS = 16
_ALPHA = 0.1
_THETA = 0.5

_NC = 2
_NS = 16
_LANES = 16
_NW = _NC * _NS
_CHUNK = 128
_NSTEPS = 80
_NSTAGE = 5
_HSTEPS = _NSTEPS // _NSTAGE
_EP = _NW * _NSTEPS * _CHUNK
_AGG_ROWS = 10112
_STRIPE = _AGG_ROWS // _NS
_ZR = 32

_BR = 1000


def _sc_edge_scatter(hs, srcw, dstw):
    mesh = plsc.VectorSubcoreMesh(core_axis_name="c", subcore_axis_name="s")

    def body(hs_hbm, srcw_hbm, dstw_hbm, out_hbm,
             agg_sh, idx_sA, idx_dA, idx_sB, idx_dB, rows0, rows1, zbuf,
             gsem0, gsem1, isem_s, isem_d):
        cid = lax.axis_index("c")
        sid = lax.axis_index("s")
        wid = cid * _NS + sid

        def prefetch(stage, s_ref, d_ref):
            pltpu.async_copy(srcw_hbm.at[wid, pl.ds(stage * _HSTEPS, _HSTEPS)],
                             s_ref, isem_s)
            pltpu.async_copy(dstw_hbm.at[wid, pl.ds(stage * _HSTEPS, _HSTEPS)],
                             d_ref, isem_d)

        def prefetch_wait(s_ref, d_ref):
            pltpu.make_async_copy(srcw_hbm.at[wid, pl.ds(0, _HSTEPS)], s_ref, isem_s).wait()
            pltpu.make_async_copy(dstw_hbm.at[wid, pl.ds(0, _HSTEPS)], d_ref, isem_d).wait()

        pltpu.sync_copy(srcw_hbm.at[wid, pl.ds(0, _HSTEPS)], idx_sA)
        pltpu.sync_copy(dstw_hbm.at[wid, pl.ds(0, _HSTEPS)], idx_dA)
        prefetch(1, idx_sB, idx_dB)
        pltpu.async_copy(hs_hbm.at[idx_sA.at[0]], rows0, gsem0)

        def zrow(r, carry):
            for c in range(_HID // _LANES):
                zbuf[r, pl.ds(c * _LANES, _LANES)] = jnp.zeros((_LANES,), jnp.float32)
            return carry
        lax.fori_loop(0, _ZR, zrow, 0)
        full, rem = divmod(_STRIPE, _ZR)
        for k in range(full):
            pltpu.sync_copy(zbuf, agg_sh.at[pl.ds(sid * _STRIPE + k * _ZR, _ZR)])
        if rem:
            pltpu.sync_copy(zbuf.at[pl.ds(0, rem)],
                            agg_sh.at[pl.ds(sid * _STRIPE + full * _ZR, rem)])
        plsc.subcore_barrier()

        npair = _HSTEPS // 2
        for stage in range(_NSTAGE):
            cur_s, cur_d = (idx_sA, idx_dA) if stage % 2 == 0 else (idx_sB, idx_dB)
            nxt_s, nxt_d = (idx_sB, idx_dB) if stage % 2 == 0 else (idx_sA, idx_dA)
            if 0 < stage < _NSTAGE - 1:
                prefetch(stage + 1, nxt_s, nxt_d)

            def step(i, carry, cur_s=cur_s, cur_d=cur_d):
                pltpu.make_async_copy(hs_hbm.at[pl.ds(0, _CHUNK)], rows0, gsem0).wait()
                pltpu.async_copy(hs_hbm.at[cur_s.at[2 * i + 1]], rows1, gsem1)
                pltpu.sync_copy(rows0, agg_sh.at[cur_d.at[2 * i]], add=True)
                pltpu.async_copy(hs_hbm.at[cur_s.at[2 * i + 2]], rows0, gsem0)
                pltpu.make_async_copy(hs_hbm.at[pl.ds(0, _CHUNK)], rows1, gsem1).wait()
                pltpu.sync_copy(rows1, agg_sh.at[cur_d.at[2 * i + 1]], add=True)
                return carry

            lax.fori_loop(0, npair - 1, step, 0)

            j = 2 * (npair - 1)
            pltpu.make_async_copy(hs_hbm.at[pl.ds(0, _CHUNK)], rows0, gsem0).wait()
            pltpu.async_copy(hs_hbm.at[cur_s.at[j + 1]], rows1, gsem1)
            pltpu.sync_copy(rows0, agg_sh.at[cur_d.at[j]], add=True)
            if stage < _NSTAGE - 1:
                prefetch_wait(nxt_s, nxt_d)
                pltpu.async_copy(hs_hbm.at[nxt_s.at[0]], rows0, gsem0)
            pltpu.make_async_copy(hs_hbm.at[pl.ds(0, _CHUNK)], rows1, gsem1).wait()
            pltpu.sync_copy(rows1, agg_sh.at[cur_d.at[j + 1]], add=True)
        plsc.subcore_barrier()

        pltpu.sync_copy(agg_sh.at[pl.ds(sid * _STRIPE, _STRIPE)],
                        out_hbm.at[cid, pl.ds(sid * _STRIPE, _STRIPE)])

    return pl.kernel(
        body,
        out_type=jax.ShapeDtypeStruct((_NC, _AGG_ROWS, _HID), jnp.float32),
        mesh=mesh,
        scratch_types=[
            pltpu.VMEM_SHARED((_AGG_ROWS, _HID), jnp.float32),
            pltpu.VMEM((_HSTEPS, _CHUNK), jnp.int32),
            pltpu.VMEM((_HSTEPS, _CHUNK), jnp.int32),
            pltpu.VMEM((_HSTEPS, _CHUNK), jnp.int32),
            pltpu.VMEM((_HSTEPS, _CHUNK), jnp.int32),
            pltpu.VMEM((_CHUNK, _HID), jnp.float32),
            pltpu.VMEM((_CHUNK, _HID), jnp.float32),
            pltpu.VMEM((_ZR, _HID), jnp.float32),
            pltpu.SemaphoreType.DMA,
            pltpu.SemaphoreType.DMA,
            pltpu.SemaphoreType.DMA,
            pltpu.SemaphoreType.DMA,
        ],
    )(hs, srcw, dstw)


def _tc_h0(x, W1, b1):
    def body(x_ref, w_ref, b_ref, h_ref):
        h = jnp.dot(x_ref[...], w_ref[...], preferred_element_type=jnp.float32)
        h_ref[...] = jnp.maximum(h + b_ref[...], 0.0)

    return pl.pallas_call(
        body,
        grid=(_N // _BR,),
        in_specs=[
            pl.BlockSpec((_BR, _M), lambda i: (i, 0)),
            pl.BlockSpec((_M, _HID), lambda i: (0, 0)),
            pl.BlockSpec((1, _HID), lambda i: (0, 0)),
        ],
        out_specs=pl.BlockSpec((_BR, _HID), lambda i: (i, 0)),
        out_shape=jax.ShapeDtypeStruct((_N, _HID), jnp.float32),
    )(x, W1, b1.reshape(1, _HID))


def _tc_scale(h0, D):
    def body(h_ref, d_ref, hs_ref, dinv_ref):
        deg = d_ref[0] + d_ref[1] + 1.0
        dinv = lax.rsqrt(deg)
        hs_ref[...] = h_ref[...] * dinv
        dinv_ref[...] = dinv

    return pl.pallas_call(
        body,
        grid=(_N // _BR,),
        in_specs=[
            pl.BlockSpec((_BR, _HID), lambda i: (i, 0)),
            pl.BlockSpec((_NC, _BR, _HID), lambda i: (0, i, 0)),
        ],
        out_specs=[
            pl.BlockSpec((_BR, _HID), lambda i: (i, 0)),
            pl.BlockSpec((_BR, _HID), lambda i: (i, 0)),
        ],
        out_shape=[
            jax.ShapeDtypeStruct((_N, _HID), jnp.float32),
            jax.ShapeDtypeStruct((_N, _HID), jnp.float32),
        ],
    )(h0, D)


def _tc_layer(S, hs, h0, dinv, weff_i):
    def body(s_ref, hs_ref, h0_ref, dinv_ref, w_ref, h_ref, hs_out_ref):
        ssum = s_ref[0] + s_ref[1] + hs_ref[...]
        supp = (1.0 - _ALPHA) * dinv_ref[...] * ssum + _ALPHA * h0_ref[...]
        h = jnp.dot(supp, w_ref[...], preferred_element_type=jnp.float32)
        h = jnp.maximum(h, 0.0)
        h_ref[...] = h
        hs_out_ref[...] = h * dinv_ref[...]

    return pl.pallas_call(
        body,
        grid=(_N // _BR,),
        in_specs=[
            pl.BlockSpec((_NC, _BR, _HID), lambda i: (0, i, 0)),
            pl.BlockSpec((_BR, _HID), lambda i: (i, 0)),
            pl.BlockSpec((_BR, _HID), lambda i: (i, 0)),
            pl.BlockSpec((_BR, _HID), lambda i: (i, 0)),
            pl.BlockSpec((_HID, _HID), lambda i: (0, 0)),
        ],
        out_specs=[
            pl.BlockSpec((_BR, _HID), lambda i: (i, 0)),
            pl.BlockSpec((_BR, _HID), lambda i: (i, 0)),
        ],
        out_shape=[
            jax.ShapeDtypeStruct((_N, _HID), jnp.float32),
            jax.ShapeDtypeStruct((_N, _HID), jnp.float32),
        ],
    )(S, hs, h0, dinv, weff_i)


def _tc_final(h, W2, b2):
    def body(h_ref, w_ref, b_ref, o_ref):
        o = jnp.dot(h_ref[...], w_ref[...], preferred_element_type=jnp.float32)
        o_ref[...] = o + b_ref[...]

    return pl.pallas_call(
        body,
        grid=(_N // _BR,),
        in_specs=[
            pl.BlockSpec((_BR, _HID), lambda i: (i, 0)),
            pl.BlockSpec((_HID, _MY), lambda i: (0, 0)),
            pl.BlockSpec((1, _MY), lambda i: (0, 0)),
        ],
        out_specs=pl.BlockSpec((_BR, _MY), lambda i: (i, 0)),
        out_shape=jax.ShapeDtypeStruct((_N, _MY), jnp.float32),
    )(h, W2, b2.reshape(1, _MY))


def kernel(x, edge_index, W1, b1, conv_w, W2, b2):
    src = edge_index[0].astype(jnp.int32)
    dst = edge_index[1].astype(jnp.int32)
    pad = _EP - _E
    pad_src = jnp.arange(pad, dtype=jnp.int32) % _N
    pad_dst = _N + (jnp.arange(pad, dtype=jnp.int32) % (_AGG_ROWS - _N))
    srcw = jnp.concatenate([src, pad_src]).reshape(_NW, _NSTEPS, _CHUNK)
    dstw = jnp.concatenate([dst, pad_dst]).reshape(_NW, _NSTEPS, _CHUNK)

    eye = jnp.eye(_HID, dtype=jnp.float32)
    betas = [float(np.log(_THETA / (i + 1) + 1.0)) for i in range(_LAYERS)]
    weff = [(1.0 - b) * eye + b * conv_w[i] for i, b in enumerate(betas)]

    ones = jnp.ones((_N, _HID), jnp.float32)
    D = _sc_edge_scatter(ones, srcw, dstw)
    h0 = _tc_h0(x, W1, b1)
    hs, dinv = _tc_scale(h0, D)

    h = h0
    for i in range(_LAYERS):
        S = _sc_edge_scatter(hs, srcw, dstw)
        h, hs = _tc_layer(S, hs, h0, dinv, weff[i])
    return _tc_final(h, W2, b2)


# speedup vs baseline: 1.3156x; 1.0409x over previous
import functools

import jax
import jax.numpy as jnp
import numpy as np
from jax import lax
from jax.experimental import pallas as pl
from jax.experimental.pallas import tpu as pltpu
from jax.experimental.pallas import tpu_sc as plsc

_N = 10000
_E = 320000
_M = 128
_HID = 128
_MY = 40
_LAYERS = 16
_ALPHA = 0.1
_THETA = 0.5

_NC = 2
_NS = 16
_LANES = 16
_NW = _NC * _NS
_CHUNK = 128
_NSTEPS = 80
_NSTAGE = 5
_HSTEPS = _NSTEPS // _NSTAGE
_EP = _NW * _NSTEPS * _CHUNK
_AGG_ROWS = 10112
_STRIPE = _AGG_ROWS // _NS
_ZR = 32

_BR = 1000


def _sc_edge_scatter(hs, srcw, dstw):
    mesh = plsc.VectorSubcoreMesh(core_axis_name="c", subcore_axis_name="s")

    def body(hs_hbm, srcw_hbm, dstw_hbm, out_hbm,
             agg_sh, idx_sA, idx_dA, idx_sB, idx_dB, rows0, rows1, zbuf,
             gsem0, gsem1, isem_s, isem_d):
        cid = lax.axis_index("c")
        sid = lax.axis_index("s")
        wid = cid * _NS + sid

        def prefetch(stage, s_ref, d_ref):
            pltpu.async_copy(srcw_hbm.at[wid, pl.ds(stage * _HSTEPS, _HSTEPS)],
                             s_ref, isem_s)
            pltpu.async_copy(dstw_hbm.at[wid, pl.ds(stage * _HSTEPS, _HSTEPS)],
                             d_ref, isem_d)

        def prefetch_wait(s_ref, d_ref):
            pltpu.make_async_copy(srcw_hbm.at[wid, pl.ds(0, _HSTEPS)], s_ref, isem_s).wait()
            pltpu.make_async_copy(dstw_hbm.at[wid, pl.ds(0, _HSTEPS)], d_ref, isem_d).wait()

        pltpu.sync_copy(srcw_hbm.at[wid, pl.ds(0, _HSTEPS)], idx_sA)
        pltpu.sync_copy(dstw_hbm.at[wid, pl.ds(0, _HSTEPS)], idx_dA)
        prefetch(1, idx_sB, idx_dB)
        pltpu.async_copy(hs_hbm.at[idx_sA.at[0]], rows0, gsem0)
        pltpu.async_copy(hs_hbm.at[idx_sA.at[1]], rows1, gsem1)

        def zrow(r, carry):
            for c in range(_HID // _LANES):
                zbuf[r, pl.ds(c * _LANES, _LANES)] = jnp.zeros((_LANES,), jnp.float32)
            return carry
        lax.fori_loop(0, _ZR, zrow, 0)
        full, rem = divmod(_STRIPE, _ZR)
        for k in range(full):
            pltpu.sync_copy(zbuf, agg_sh.at[pl.ds(sid * _STRIPE + k * _ZR, _ZR)])
        if rem:
            pltpu.sync_copy(zbuf.at[pl.ds(0, rem)],
                            agg_sh.at[pl.ds(sid * _STRIPE + full * _ZR, rem)])
        plsc.subcore_barrier()

        npair = _HSTEPS // 2
        for stage in range(_NSTAGE):
            cur_s, cur_d = (idx_sA, idx_dA) if stage % 2 == 0 else (idx_sB, idx_dB)
            nxt_s, nxt_d = (idx_sB, idx_dB) if stage % 2 == 0 else (idx_sA, idx_dA)
            if 0 < stage < _NSTAGE - 1:
                prefetch(stage + 1, nxt_s, nxt_d)

            def step(i, carry, cur_s=cur_s, cur_d=cur_d):
                pltpu.make_async_copy(hs_hbm.at[pl.ds(0, _CHUNK)], rows0, gsem0).wait()
                pltpu.sync_copy(rows0, agg_sh.at[cur_d.at[2 * i]], add=True)
                pltpu.async_copy(hs_hbm.at[cur_s.at[2 * i + 2]], rows0, gsem0)
                pltpu.make_async_copy(hs_hbm.at[pl.ds(0, _CHUNK)], rows1, gsem1).wait()
                pltpu.sync_copy(rows1, agg_sh.at[cur_d.at[2 * i + 1]], add=True)
                pltpu.async_copy(hs_hbm.at[cur_s.at[2 * i + 3]], rows1, gsem1)
                return carry

            lax.fori_loop(0, npair - 1, step, 0)

            j = 2 * (npair - 1)
            pltpu.make_async_copy(hs_hbm.at[pl.ds(0, _CHUNK)], rows0, gsem0).wait()
            pltpu.sync_copy(rows0, agg_sh.at[cur_d.at[j]], add=True)
            if stage < _NSTAGE - 1:
                prefetch_wait(nxt_s, nxt_d)
                pltpu.async_copy(hs_hbm.at[nxt_s.at[0]], rows0, gsem0)
            pltpu.make_async_copy(hs_hbm.at[pl.ds(0, _CHUNK)], rows1, gsem1).wait()
            pltpu.sync_copy(rows1, agg_sh.at[cur_d.at[j + 1]], add=True)
            if stage < _NSTAGE - 1:
                pltpu.async_copy(hs_hbm.at[nxt_s.at[1]], rows1, gsem1)
        plsc.subcore_barrier()

        pltpu.sync_copy(agg_sh.at[pl.ds(sid * _STRIPE, _STRIPE)],
                        out_hbm.at[cid, pl.ds(sid * _STRIPE, _STRIPE)])

    return pl.kernel(
        body,
        out_type=jax.ShapeDtypeStruct((_NC, _AGG_ROWS, _HID), jnp.float32),
        mesh=mesh,
        scratch_types=[
            pltpu.VMEM_SHARED((_AGG_ROWS, _HID), jnp.float32),
            pltpu.VMEM((_HSTEPS, _CHUNK), jnp.int32),
            pltpu.VMEM((_HSTEPS, _CHUNK), jnp.int32),
            pltpu.VMEM((_HSTEPS, _CHUNK), jnp.int32),
            pltpu.VMEM((_HSTEPS, _CHUNK), jnp.int32),
            pltpu.VMEM((_CHUNK, _HID), jnp.float32),
            pltpu.VMEM((_CHUNK, _HID), jnp.float32),
            pltpu.VMEM((_ZR, _HID), jnp.float32),
            pltpu.SemaphoreType.DMA,
            pltpu.SemaphoreType.DMA,
            pltpu.SemaphoreType.DMA,
            pltpu.SemaphoreType.DMA,
        ],
    )(hs, srcw, dstw)


def _tc_h0(x, W1, b1):
    def body(x_ref, w_ref, b_ref, h_ref):
        h = jnp.dot(x_ref[...], w_ref[...], preferred_element_type=jnp.float32)
        h_ref[...] = jnp.maximum(h + b_ref[...], 0.0)

    return pl.pallas_call(
        body,
        grid=(_N // _BR,),
        in_specs=[
            pl.BlockSpec((_BR, _M), lambda i: (i, 0)),
            pl.BlockSpec((_M, _HID), lambda i: (0, 0)),
            pl.BlockSpec((1, _HID), lambda i: (0, 0)),
        ],
        out_specs=pl.BlockSpec((_BR, _HID), lambda i: (i, 0)),
        out_shape=jax.ShapeDtypeStruct((_N, _HID), jnp.float32),
    )(x, W1, b1.reshape(1, _HID))


def _tc_scale(h0, D):
    def body(h_ref, d_ref, hs_ref, dinv_ref):
        deg = d_ref[0] + d_ref[1] + 1.0
        dinv = lax.rsqrt(deg)
        hs_ref[...] = h_ref[...] * dinv
        dinv_ref[...] = dinv

    return pl.pallas_call(
        body,
        grid=(_N // _BR,),
        in_specs=[
            pl.BlockSpec((_BR, _HID), lambda i: (i, 0)),
            pl.BlockSpec((_NC, _BR, _HID), lambda i: (0, i, 0)),
        ],
        out_specs=[
            pl.BlockSpec((_BR, _HID), lambda i: (i, 0)),
            pl.BlockSpec((_BR, _HID), lambda i: (i, 0)),
        ],
        out_shape=[
            jax.ShapeDtypeStruct((_N, _HID), jnp.float32),
            jax.ShapeDtypeStruct((_N, _HID), jnp.float32),
        ],
    )(h0, D)


def _tc_layer(S, hs, h0, dinv, weff_i):
    def body(s_ref, hs_ref, h0_ref, dinv_ref, w_ref, h_ref, hs_out_ref):
        ssum = s_ref[0] + s_ref[1] + hs_ref[...]
        supp = (1.0 - _ALPHA) * dinv_ref[...] * ssum + _ALPHA * h0_ref[...]
        h = jnp.dot(supp, w_ref[...], preferred_element_type=jnp.float32)
        h = jnp.maximum(h, 0.0)
        h_ref[...] = h
        hs_out_ref[...] = h * dinv_ref[...]

    return pl.pallas_call(
        body,
        grid=(_N // _BR,),
        in_specs=[
            pl.BlockSpec((_NC, _BR, _HID), lambda i: (0, i, 0)),
            pl.BlockSpec((_BR, _HID), lambda i: (i, 0)),
            pl.BlockSpec((_BR, _HID), lambda i: (i, 0)),
            pl.BlockSpec((_BR, _HID), lambda i: (i, 0)),
            pl.BlockSpec((_HID, _HID), lambda i: (0, 0)),
        ],
        out_specs=[
            pl.BlockSpec((_BR, _HID), lambda i: (i, 0)),
            pl.BlockSpec((_BR, _HID), lambda i: (i, 0)),
        ],
        out_shape=[
            jax.ShapeDtypeStruct((_N, _HID), jnp.float32),
            jax.ShapeDtypeStruct((_N, _HID), jnp.float32),
        ],
    )(S, hs, h0, dinv, weff_i)


def _tc_final(h, W2, b2):
    def body(h_ref, w_ref, b_ref, o_ref):
        o = jnp.dot(h_ref[...], w_ref[...], preferred_element_type=jnp.float32)
        o_ref[...] = o + b_ref[...]

    return pl.pallas_call(
        body,
        grid=(_N // _BR,),
        in_specs=[
            pl.BlockSpec((_BR, _HID), lambda i: (i, 0)),
            pl.BlockSpec((_HID, _MY), lambda i: (0, 0)),
            pl.BlockSpec((1, _MY), lambda i: (0, 0)),
        ],
        out_specs=pl.BlockSpec((_BR, _MY), lambda i: (i, 0)),
        out_shape=jax.ShapeDtypeStruct((_N, _MY), jnp.float32),
    )(h, W2, b2.reshape(1, _MY))


def kernel(x, edge_index, W1, b1, conv_w, W2, b2):
    src = edge_index[0].astype(jnp.int32)
    dst = edge_index[1].astype(jnp.int32)
    pad = _EP - _E
    pad_src = jnp.arange(pad, dtype=jnp.int32) % _N
    pad_dst = _N + (jnp.arange(pad, dtype=jnp.int32) % (_AGG_ROWS - _N))
    srcw = jnp.concatenate([src, pad_src]).reshape(_NW, _NSTEPS, _CHUNK)
    dstw = jnp.concatenate([dst, pad_dst]).reshape(_NW, _NSTEPS, _CHUNK)

    eye = jnp.eye(_HID, dtype=jnp.float32)
    betas = [float(np.log(_THETA / (i + 1) + 1.0)) for i in range(_LAYERS)]
    weff = [(1.0 - b) * eye + b * conv_w[i] for i, b in enumerate(betas)]

    ones = jnp.ones((_N, _HID), jnp.float32)
    D = _sc_edge_scatter(ones, srcw, dstw)
    h0 = _tc_h0(x, W1, b1)
    hs, dinv = _tc_scale(h0, D)

    h = h0
    for i in range(_LAYERS):
        S = _sc_edge_scatter(hs, srcw, dstw)
        h, hs = _tc_layer(S, hs, h0, dinv, weff[i])
    return _tc_final(h, W2, b2)
